# baseline (speedup 1.0000x reference)
import jax
import jax.numpy as jnp
from jax.experimental import pallas as pl
from jax.experimental.pallas import tpu as pltpu

_BATCH = 128
_F = 2048
_OUT = 2048
_CMS = 32
_U = 64
_TILE = 256
_GRID = _OUT // _TILE


def _gumbel_traced():
    base = jax.random.key(42)
    keys = jax.vmap(jax.random.fold_in, in_axes=(None, 0))(base, jnp.arange(_CMS))
    return jax.vmap(lambda k: jax.random.gumbel(k, (_BATCH, _U), jnp.float32))(keys)


def _body(x_ref, w_ref, b_ref, g_ref, out_ref, xt_ref, zs_ref):
    k = pl.program_id(0)

    @pl.when(k == 0)
    def _stage():
        xt_ref[...] = jnp.transpose(x_ref[...])

    zt = jax.lax.dot_general(
        w_ref[...], xt_ref[...], (((1,), (0,)), ((), ())),
        preferred_element_type=jnp.float32)
    na = jnp.sum(xt_ref[...], axis=0, keepdims=True)
    zs_ref[pl.ds(k * _TILE, _TILE), :] = (zt + b_ref[...]) / na

    @pl.when(k == _GRID - 1)
    def _sample():
        z = zs_ref[...]
        m = jnp.max(z, axis=0, keepdims=True)
        row_cm = jax.lax.broadcasted_iota(jnp.int32, (_OUT, _BATCH), 0) // _U
        lane = jax.lax.broadcasted_iota(jnp.int32, (_OUT, _BATCH), 1)
        sel = (row_cm == lane).astype(jnp.float32)
        mrows = jax.lax.dot_general(
            sel, m, (((1,), (1,)), ((), ())),
            preferred_element_type=jnp.float32)
        gt = jnp.transpose(g_ref[...], (0, 2, 1))
        s3 = jnp.exp(z - mrows).reshape(_CMS, _U, _BATCH) + gt
        gm = jnp.max(s3, axis=1, keepdims=True)
        sub = jax.lax.broadcasted_iota(jnp.int32, (_CMS, _U, _BATCH), 1)
        idx = jnp.min(jnp.where(s3 >= gm, sub, _U), axis=1)
        idx_t = jnp.transpose(idx.astype(jnp.float32))
        cm_of_col = jax.lax.broadcasted_iota(jnp.int32, (_CMS, _OUT), 1) // _U
        cm_row = jax.lax.broadcasted_iota(jnp.int32, (_CMS, _OUT), 0)
        selc = (cm_of_col == cm_row).astype(jnp.float32)
        chosen = jax.lax.dot_general(
            idx_t, selc, (((1,), (0,)), ((), ())),
            preferred_element_type=jnp.float32)
        unit = (jax.lax.broadcasted_iota(jnp.int32, (_BATCH, _OUT), 1) % _U
                ).astype(jnp.float32)
        out_ref[...] = (unit == chosen).astype(jnp.float32)


@jax.jit
def _impl(x, W_in, b_in):
    b2 = b_in.reshape(_OUT, 1)
    g = _gumbel_traced()
    return pl.pallas_call(
        _body,
        grid=(_GRID,),
        in_specs=[
            pl.BlockSpec((_BATCH, _F), lambda k: (0, 0)),
            pl.BlockSpec((_TILE, _F), lambda k: (k, 0)),
            pl.BlockSpec((_TILE, 1), lambda k: (k, 0)),
            pl.BlockSpec((_CMS, _BATCH, _U), lambda k: (0, 0, 0)),
        ],
        out_specs=pl.BlockSpec((_BATCH, _OUT), lambda k: (0, 0)),
        out_shape=jax.ShapeDtypeStruct((_BATCH, _OUT), jnp.float32),
        scratch_shapes=[
            pltpu.VMEM((_F, _BATCH), jnp.float32),
            pltpu.VMEM((_OUT, _BATCH), jnp.float32),
        ],
    )(x, W_in, b2, g)


def kernel(x, W_in, b_in):
    return _impl(x, W_in, b_in)


# speedup vs baseline: 1.6261x; 1.6261x over previous
import jax
import jax.numpy as jnp
import numpy as np
from jax.experimental import pallas as pl
from jax.experimental.pallas import tpu as pltpu

_BATCH = 128
_F = 2048
_OUT = 2048
_CMS = 32
_U = 64
_TILE = 256
_GRID = _OUT // _TILE


def _gumbel_traced():
    base = jax.random.key(42)
    keys = jax.vmap(jax.random.fold_in, in_axes=(None, 0))(base, jnp.arange(_CMS))
    return jax.vmap(lambda k: jax.random.gumbel(k, (_BATCH, _U), jnp.float32))(keys)


_NOISE = None


def _noise():
    global _NOISE
    if _NOISE is None:
        with jax.ensure_compile_time_eval():
            _NOISE = np.asarray(_gumbel_traced())
    return _NOISE


def _body(x_ref, w_ref, b_ref, g_ref, out_ref, xt_ref, zs_ref):
    k = pl.program_id(0)

    @pl.when(k == 0)
    def _stage():
        xt_ref[...] = jnp.transpose(x_ref[...])

    zt = jax.lax.dot_general(
        w_ref[...], xt_ref[...], (((1,), (0,)), ((), ())),
        preferred_element_type=jnp.float32)
    na = jnp.sum(xt_ref[...], axis=0, keepdims=True)
    zs_ref[pl.ds(k * _TILE, _TILE), :] = (zt + b_ref[...]) / na

    @pl.when(k == _GRID - 1)
    def _sample():
        z = zs_ref[...]
        m = jnp.max(z, axis=0, keepdims=True)
        row_cm = jax.lax.broadcasted_iota(jnp.int32, (_OUT, _BATCH), 0) // _U
        lane = jax.lax.broadcasted_iota(jnp.int32, (_OUT, _BATCH), 1)
        sel = (row_cm == lane).astype(jnp.float32)
        mrows = jax.lax.dot_general(
            sel, m, (((1,), (1,)), ((), ())),
            preferred_element_type=jnp.float32)
        gt = jnp.transpose(g_ref[...], (0, 2, 1))
        s3 = jnp.exp(z - mrows).reshape(_CMS, _U, _BATCH) + gt
        gm = jnp.max(s3, axis=1, keepdims=True)
        sub = jax.lax.broadcasted_iota(jnp.int32, (_CMS, _U, _BATCH), 1)
        idx = jnp.min(jnp.where(s3 >= gm, sub, _U), axis=1)
        idx_t = jnp.transpose(idx.astype(jnp.float32))
        cm_of_col = jax.lax.broadcasted_iota(jnp.int32, (_CMS, _OUT), 1) // _U
        cm_row = jax.lax.broadcasted_iota(jnp.int32, (_CMS, _OUT), 0)
        selc = (cm_of_col == cm_row).astype(jnp.float32)
        chosen = jax.lax.dot_general(
            idx_t, selc, (((1,), (0,)), ((), ())),
            preferred_element_type=jnp.float32)
        unit = (jax.lax.broadcasted_iota(jnp.int32, (_BATCH, _OUT), 1) % _U
                ).astype(jnp.float32)
        out_ref[...] = (unit == chosen).astype(jnp.float32)


@jax.jit
def _impl(x, W_in, b_in):
    b2 = b_in.reshape(_OUT, 1)
    g = jnp.asarray(_noise())
    return pl.pallas_call(
        _body,
        grid=(_GRID,),
        in_specs=[
            pl.BlockSpec((_BATCH, _F), lambda k: (0, 0)),
            pl.BlockSpec((_TILE, _F), lambda k: (k, 0)),
            pl.BlockSpec((_TILE, 1), lambda k: (k, 0)),
            pl.BlockSpec((_CMS, _BATCH, _U), lambda k: (0, 0, 0)),
        ],
        out_specs=pl.BlockSpec((_BATCH, _OUT), lambda k: (0, 0)),
        out_shape=jax.ShapeDtypeStruct((_BATCH, _OUT), jnp.float32),
        scratch_shapes=[
            pltpu.VMEM((_F, _BATCH), jnp.float32),
            pltpu.VMEM((_OUT, _BATCH), jnp.float32),
        ],
    )(x, W_in, b2, g)


def kernel(x, W_in, b_in):
    return _impl(x, W_in, b_in)


# speedup vs baseline: 1.7702x; 1.0886x over previous
import jax
import jax.numpy as jnp
import numpy as np
from jax.experimental import pallas as pl
from jax.experimental.pallas import tpu as pltpu

_BATCH = 128
_F = 2048
_OUT = 2048
_CMS = 32
_U = 64
_TILE = 512
_GRID = _OUT // _TILE


def _gumbel_traced():
    base = jax.random.key(42)
    keys = jax.vmap(jax.random.fold_in, in_axes=(None, 0))(base, jnp.arange(_CMS))
    return jax.vmap(lambda k: jax.random.gumbel(k, (_BATCH, _U), jnp.float32))(keys)


_NOISE = None


def _noise():
    global _NOISE
    if _NOISE is None:
        with jax.ensure_compile_time_eval():
            _NOISE = np.asarray(_gumbel_traced())
    return _NOISE


def _body(x_ref, w_ref, b_ref, g_ref, out_ref, xt_ref, zs_ref):
    k = pl.program_id(0)

    @pl.when(k == 0)
    def _stage():
        xt_ref[...] = jnp.transpose(x_ref[...])

    zt = jax.lax.dot_general(
        w_ref[...], xt_ref[...], (((1,), (0,)), ((), ())),
        preferred_element_type=jnp.float32)
    na = jnp.sum(xt_ref[...], axis=0, keepdims=True)
    zs_ref[pl.ds(k * _TILE, _TILE), :] = (zt + b_ref[...]) / na

    @pl.when(k == _GRID - 1)
    def _sample():
        z = zs_ref[...]
        m = jnp.max(z, axis=0, keepdims=True)
        row_cm = jax.lax.broadcasted_iota(jnp.int32, (_OUT, _BATCH), 0) // _U
        lane = jax.lax.broadcasted_iota(jnp.int32, (_OUT, _BATCH), 1)
        sel = (row_cm == lane).astype(jnp.float32)
        mrows = jax.lax.dot_general(
            sel, m, (((1,), (1,)), ((), ())),
            preferred_element_type=jnp.float32)
        gt = jnp.transpose(g_ref[...], (0, 2, 1))
        s3 = jnp.exp(z - mrows).reshape(_CMS, _U, _BATCH) + gt
        gm = jnp.max(s3, axis=1, keepdims=True)
        sub = jax.lax.broadcasted_iota(jnp.int32, (_CMS, _U, _BATCH), 1)
        idx = jnp.min(jnp.where(s3 >= gm, sub, _U), axis=1)
        idx_t = jnp.transpose(idx.astype(jnp.float32))
        cm_of_col = jax.lax.broadcasted_iota(jnp.int32, (_CMS, _OUT), 1) // _U
        cm_row = jax.lax.broadcasted_iota(jnp.int32, (_CMS, _OUT), 0)
        selc = (cm_of_col == cm_row).astype(jnp.float32)
        chosen = jax.lax.dot_general(
            idx_t, selc, (((1,), (0,)), ((), ())),
            preferred_element_type=jnp.float32)
        unit = (jax.lax.broadcasted_iota(jnp.int32, (_BATCH, _OUT), 1) % _U
                ).astype(jnp.float32)
        out_ref[...] = (unit == chosen).astype(jnp.float32)


@jax.jit
def _impl(x, W_in, b_in):
    b2 = b_in.reshape(_OUT, 1)
    g = jnp.asarray(_noise())
    return pl.pallas_call(
        _body,
        grid=(_GRID,),
        in_specs=[
            pl.BlockSpec((_BATCH, _F), lambda k: (0, 0)),
            pl.BlockSpec((_TILE, _F), lambda k: (k, 0)),
            pl.BlockSpec((_TILE, 1), lambda k: (k, 0)),
            pl.BlockSpec((_CMS, _BATCH, _U), lambda k: (0, 0, 0)),
        ],
        out_specs=pl.BlockSpec((_BATCH, _OUT), lambda k: (0, 0)),
        out_shape=jax.ShapeDtypeStruct((_BATCH, _OUT), jnp.float32),
        scratch_shapes=[
            pltpu.VMEM((_F, _BATCH), jnp.float32),
            pltpu.VMEM((_OUT, _BATCH), jnp.float32),
        ],
    )(x, W_in, b2, g)


def kernel(x, W_in, b_in):
    return _impl(x, W_in, b_in)


# speedup vs baseline: 1.7763x; 1.0035x over previous
import jax
import jax.numpy as jnp
import numpy as np
from jax.experimental import pallas as pl
from jax.experimental.pallas import tpu as pltpu

_BATCH = 128
_F = 2048
_OUT = 2048
_CMS = 32
_U = 64
_TILE = 1024
_GRID = _OUT // _TILE


def _gumbel_traced():
    base = jax.random.key(42)
    keys = jax.vmap(jax.random.fold_in, in_axes=(None, 0))(base, jnp.arange(_CMS))
    return jax.vmap(lambda k: jax.random.gumbel(k, (_BATCH, _U), jnp.float32))(keys)


_NOISE = None


def _noise():
    global _NOISE
    if _NOISE is None:
        with jax.ensure_compile_time_eval():
            _NOISE = np.asarray(_gumbel_traced())
    return _NOISE


def _body(x_ref, w_ref, b_ref, g_ref, out_ref, xt_ref, zs_ref):
    k = pl.program_id(0)

    @pl.when(k == 0)
    def _stage():
        xt_ref[...] = jnp.transpose(x_ref[...])

    zt = jax.lax.dot_general(
        w_ref[...], xt_ref[...], (((1,), (0,)), ((), ())),
        preferred_element_type=jnp.float32)
    na = jnp.sum(xt_ref[...], axis=0, keepdims=True)
    zs_ref[pl.ds(k * _TILE, _TILE), :] = (zt + b_ref[...]) / na

    @pl.when(k == _GRID - 1)
    def _sample():
        z = zs_ref[...]
        m = jnp.max(z, axis=0, keepdims=True)
        row_cm = jax.lax.broadcasted_iota(jnp.int32, (_OUT, _BATCH), 0) // _U
        lane = jax.lax.broadcasted_iota(jnp.int32, (_OUT, _BATCH), 1)
        sel = (row_cm == lane).astype(jnp.float32)
        mrows = jax.lax.dot_general(
            sel, m, (((1,), (1,)), ((), ())),
            preferred_element_type=jnp.float32)
        gt = jnp.transpose(g_ref[...], (0, 2, 1))
        s3 = jnp.exp(z - mrows).reshape(_CMS, _U, _BATCH) + gt
        gm = jnp.max(s3, axis=1, keepdims=True)
        sub = jax.lax.broadcasted_iota(jnp.int32, (_CMS, _U, _BATCH), 1)
        idx = jnp.min(jnp.where(s3 >= gm, sub, _U), axis=1)
        idx_t = jnp.transpose(idx.astype(jnp.float32))
        cm_of_col = jax.lax.broadcasted_iota(jnp.int32, (_CMS, _OUT), 1) // _U
        cm_row = jax.lax.broadcasted_iota(jnp.int32, (_CMS, _OUT), 0)
        selc = (cm_of_col == cm_row).astype(jnp.float32)
        chosen = jax.lax.dot_general(
            idx_t, selc, (((1,), (0,)), ((), ())),
            preferred_element_type=jnp.float32)
        unit = (jax.lax.broadcasted_iota(jnp.int32, (_BATCH, _OUT), 1) % _U
                ).astype(jnp.float32)
        out_ref[...] = (unit == chosen).astype(jnp.float32)


@jax.jit
def _impl(x, W_in, b_in):
    b2 = b_in.reshape(_OUT, 1)
    g = jnp.asarray(_noise())
    return pl.pallas_call(
        _body,
        grid=(_GRID,),
        in_specs=[
            pl.BlockSpec((_BATCH, _F), lambda k: (0, 0)),
            pl.BlockSpec((_TILE, _F), lambda k: (k, 0)),
            pl.BlockSpec((_TILE, 1), lambda k: (k, 0)),
            pl.BlockSpec((_CMS, _BATCH, _U), lambda k: (0, 0, 0)),
        ],
        out_specs=pl.BlockSpec((_BATCH, _OUT), lambda k: (0, 0)),
        out_shape=jax.ShapeDtypeStruct((_BATCH, _OUT), jnp.float32),
        scratch_shapes=[
            pltpu.VMEM((_F, _BATCH), jnp.float32),
            pltpu.VMEM((_OUT, _BATCH), jnp.float32),
        ],
    )(x, W_in, b2, g)


def kernel(x, W_in, b_in):
    return _impl(x, W_in, b_in)
